# baseline (speedup 1.0000x reference)
import functools

import jax
import jax.numpy as jnp
from jax import lax
from jax.experimental import pallas as pl
from jax.experimental.pallas import tpu as pltpu
from jax.experimental.pallas import tpu_sc as plsc

N = 10000
E = 320000
B = 128
NB = E // B
NBP = 2560
NC = 2
NS = 16
OWN = 624
TAIL = N - NS * OWN
CHUNKS = (128, 128, 128, 128, 112)


def _fill(ref, nrows, ncols, val):
    nc16 = ncols // 16

    def body(i, carry):
        r = i // nc16
        k = i % nc16
        ref[r, pl.ds(k * 16, 16)] = jnp.full((16,), val, jnp.float32)
        return carry

    lax.fori_loop(0, nrows * nc16, body, 0)


def _fill_i32(ref, val):
    for j in range(B // 16):
        ref[pl.ds(j * 16, 16)] = jnp.full((16,), val, jnp.int32)


def _acc_zero(zero_v, acc, s):
    base = pl.multiple_of(s * OWN, 8)
    off = 0
    for sz in CHUNKS:
        pltpu.sync_copy(zero_v.at[pl.ds(0, sz)], acc.at[pl.ds(base + off, sz)])
        off += sz

    @pl.when(s == 0)
    def _():
        pltpu.sync_copy(zero_v.at[pl.ds(0, TAIL)],
                        acc.at[pl.ds(NS * OWN, TAIL)])


def _acc_writeout(acc, out_hbm, c, s):
    base = pl.multiple_of(s * OWN, 8)
    out_base = pl.multiple_of(c * N + base, 8)
    off = 0
    for sz in CHUNKS:
        pltpu.sync_copy(acc.at[pl.ds(base + off, sz)],
                        out_hbm.at[pl.ds(out_base + off, sz)])
        off += sz

    @pl.when(s == 0)
    def _():
        pltpu.sync_copy(acc.at[pl.ds(NS * OWN, TAIL)],
                        out_hbm.at[pl.ds(pl.multiple_of(c * N + NS * OWN, 8),
                                         TAIL)])


def _make_deg_kernel():
    mesh = plsc.VectorSubcoreMesh(core_axis_name="c", subcore_axis_name="s")
    nb_sc = NBP // NC
    n_t = nb_sc // NS

    @functools.partial(
        pl.kernel,
        mesh=mesh,
        out_type=jax.ShapeDtypeStruct((NC * N, 16), jnp.float32),
        scratch_types=[
            pltpu.VMEM((B,), jnp.int32),
            pltpu.VMEM((B, 16), jnp.float32),
            pltpu.VMEM((128, 16), jnp.float32),
            pltpu.VMEM_SHARED((N + 8, 16), jnp.float32),
        ],
    )
    def deg_kernel(dst_hbm, out_hbm, dst_v, ones_v, zero_v, acc):
        c = lax.axis_index("c")
        s = lax.axis_index("s")
        _fill(ones_v, B, 16, 1.0)
        _fill(zero_v, 128, 16, 0.0)
        _acc_zero(zero_v, acc, s)
        plsc.subcore_barrier()

        def body(t, carry):
            b = c * nb_sc + t * NS + s
            pltpu.sync_copy(dst_hbm.at[pl.ds(b * B, B)], dst_v)
            pltpu.sync_copy(ones_v, acc.at[dst_v], add=True)
            return carry

        lax.fori_loop(0, n_t, body, 0)
        plsc.subcore_barrier()
        _acc_writeout(acc, out_hbm, c, s)

    return deg_kernel


def _make_agg_kernel(chan_split):
    mesh = plsc.VectorSubcoreMesh(core_axis_name="c", subcore_axis_name="s")
    nb_sc = NBP if chan_split else NBP // NC
    n_t = nb_sc // NS

    @functools.partial(
        pl.kernel,
        mesh=mesh,
        out_type=jax.ShapeDtypeStruct((NC * N, 128), jnp.float32),
        scratch_types=[
            pltpu.VMEM((B,), jnp.int32),
            pltpu.VMEM((B,), jnp.int32),
            pltpu.VMEM((B, 128), jnp.float32),
            pltpu.SemaphoreType.DMA,
            pltpu.VMEM_SHARED((N + 8, 128), jnp.float32),
        ],
    )
    def agg_kernel(table_hbm, src_hbm, dst_hbm, out_hbm,
                   sv, dv, rows_v, sem_g, acc):
        c = lax.axis_index("c")
        s = lax.axis_index("s")
        _fill(rows_v, B, 128, 0.0)
        _acc_zero(rows_v, acc, s)
        plsc.subcore_barrier()

        def body(t, carry):
            b = (t * NS + s) if chan_split else (c * nb_sc + t * NS + s)
            pltpu.sync_copy(src_hbm.at[pl.ds(b * B, B)], sv)
            pltpu.sync_copy(dst_hbm.at[pl.ds(b * B, B)], dv)
            if chan_split:
                off = c * N
                for j in range(B // 16):
                    sv[pl.ds(j * 16, 16)] = sv[pl.ds(j * 16, 16)] + off
            pltpu.async_copy(table_hbm.at[sv], rows_v, sem_g).wait()
            pltpu.sync_copy(rows_v, acc.at[dv], add=True)
            return carry

        lax.fori_loop(0, n_t, body, 0)
        plsc.subcore_barrier()
        _acc_writeout(acc, out_hbm, c, s)

    return agg_kernel


_deg_call = _make_deg_kernel()
_agg_l1 = _make_agg_kernel(chan_split=True)
_agg_l2 = _make_agg_kernel(chan_split=False)


def _d_from_degp(degp_ref):
    deg = degp_ref[0, :, 0:1] + degp_ref[1, :, 0:1] + 1.0
    return lax.rsqrt(deg)


def _tc1_body(x_ref, w1_ref, degp_ref, p1_ref):
    h = jnp.dot(x_ref[...], w1_ref[...],
                preferred_element_type=jnp.float32,
                precision=lax.Precision.HIGHEST)
    d = _d_from_degp(degp_ref)
    p1_ref[0] = h[:, :128] * d
    p1_ref[1] = h[:, 128:] * d


def _tc2_body(agg_ref, p1_ref, degp_ref, b1_ref, w2_ref, p2_ref):
    d = _d_from_degp(degp_ref)
    s0 = jnp.maximum(d * (agg_ref[0] + p1_ref[0]) + b1_ref[:, :128], 0.0)
    s1 = jnp.maximum(d * (agg_ref[1] + p1_ref[1]) + b1_ref[:, 128:], 0.0)
    h2 = (jnp.dot(s0, w2_ref[:128, :], preferred_element_type=jnp.float32,
                  precision=lax.Precision.HIGHEST)
          + jnp.dot(s1, w2_ref[128:, :], preferred_element_type=jnp.float32,
                    precision=lax.Precision.HIGHEST))
    p2_ref[...] = d * h2


def _tc3_body(agg_ref, p2_ref, degp_ref, b2_ref, out_ref):
    d = _d_from_degp(degp_ref)
    out_ref[...] = d * (agg_ref[0] + agg_ref[1] + p2_ref[...]) + b2_ref[...]


_RB = 1000
_GRID = N // _RB

_degp_spec = pl.BlockSpec((2, _RB, 16), lambda i: (0, i, 0))


def _tc1(x, w1, degp):
    return pl.pallas_call(
        _tc1_body,
        grid=(_GRID,),
        in_specs=[
            pl.BlockSpec((_RB, 128), lambda i: (i, 0)),
            pl.BlockSpec((128, 256), lambda i: (0, 0)),
            _degp_spec,
        ],
        out_specs=pl.BlockSpec((2, _RB, 128), lambda i: (0, i, 0)),
        out_shape=jax.ShapeDtypeStruct((2, N, 128), jnp.float32),
    )(x, w1, degp)


def _tc2(agg1, p1, degp, b1, w2):
    return pl.pallas_call(
        _tc2_body,
        grid=(_GRID,),
        in_specs=[
            pl.BlockSpec((2, _RB, 128), lambda i: (0, i, 0)),
            pl.BlockSpec((2, _RB, 128), lambda i: (0, i, 0)),
            _degp_spec,
            pl.BlockSpec((1, 256), lambda i: (0, 0)),
            pl.BlockSpec((256, 128), lambda i: (0, 0)),
        ],
        out_specs=pl.BlockSpec((_RB, 128), lambda i: (i, 0)),
        out_shape=jax.ShapeDtypeStruct((N, 128), jnp.float32),
    )(agg1, p1, degp, b1, w2)


def _tc3(agg2, p2, degp, b2):
    return pl.pallas_call(
        _tc3_body,
        grid=(_GRID,),
        in_specs=[
            pl.BlockSpec((2, _RB, 128), lambda i: (0, i, 0)),
            pl.BlockSpec((_RB, 128), lambda i: (i, 0)),
            _degp_spec,
            pl.BlockSpec((1, 128), lambda i: (0, 0)),
        ],
        out_specs=pl.BlockSpec((_RB, 128), lambda i: (i, 0)),
        out_shape=jax.ShapeDtypeStruct((N, 128), jnp.float32),
    )(agg2, p2, degp, b2)


def kernel(x, edge_index, W1, b1, W2, b2):
    ei = edge_index.astype(jnp.int32)
    src, dst = ei[0], ei[1]
    pad = NBP * B - E
    srcp = jnp.concatenate([src, jnp.zeros((pad,), jnp.int32)])
    dstp = jnp.concatenate([dst, jnp.full((pad,), N, jnp.int32)])

    degp = _deg_call(dstp).reshape(2, N, 16)
    p1 = _tc1(x, W1, degp)
    agg1 = _agg_l1(p1.reshape(2 * N, 128), srcp, dstp).reshape(2, N, 128)
    p2 = _tc2(agg1, p1, degp, b1.reshape(1, 256), W2)
    agg2 = _agg_l2(p2, srcp, dstp).reshape(2, N, 128)
    return _tc3(agg2, p2, degp, b2.reshape(1, 128))


# speedup vs baseline: 1.6408x; 1.6408x over previous
import functools

import jax
import jax.numpy as jnp
from jax import lax
from jax.experimental import pallas as pl
from jax.experimental.pallas import tpu as pltpu
from jax.experimental.pallas import tpu_sc as plsc

N = 10000
E = 320000
B = 128
NB = E // B
NC = 2
NS = 16
OWN = 624
TAIL = N - NS * OWN
CHUNKS = (128, 128, 128, 128, 112)


def _fill(ref, nrows, ncols, val):
    nc16 = ncols // 16

    def body(i, carry):
        r = i // nc16
        k = i % nc16
        ref[r, pl.ds(k * 16, 16)] = jnp.full((16,), val, jnp.float32)
        return carry

    lax.fori_loop(0, nrows * nc16, body, 0)


def _fill_i32(ref, val):
    for j in range(B // 16):
        ref[pl.ds(j * 16, 16)] = jnp.full((16,), val, jnp.int32)


def _acc_zero(zero_v, acc, s):
    base = pl.multiple_of(s * OWN, 8)
    off = 0
    for sz in CHUNKS:
        pltpu.sync_copy(zero_v.at[pl.ds(0, sz)], acc.at[pl.ds(base + off, sz)])
        off += sz

    @pl.when(s == 0)
    def _():
        pltpu.sync_copy(zero_v.at[pl.ds(0, TAIL)],
                        acc.at[pl.ds(NS * OWN, TAIL)])


def _acc_writeout(acc, out_hbm, c, s):
    base = pl.multiple_of(s * OWN, 8)
    out_base = pl.multiple_of(c * N + base, 8)
    off = 0
    for sz in CHUNKS:
        pltpu.sync_copy(acc.at[pl.ds(base + off, sz)],
                        out_hbm.at[pl.ds(out_base + off, sz)])
        off += sz

    @pl.when(s == 0)
    def _():
        pltpu.sync_copy(acc.at[pl.ds(NS * OWN, TAIL)],
                        out_hbm.at[pl.ds(pl.multiple_of(c * N + NS * OWN, 8),
                                         TAIL)])


def _make_deg_kernel():
    mesh = plsc.VectorSubcoreMesh(core_axis_name="c", subcore_axis_name="s")
    nb_sc = NB // NC

    @functools.partial(
        pl.kernel,
        mesh=mesh,
        out_type=jax.ShapeDtypeStruct((NC * N, 16), jnp.float32),
        scratch_types=[
            pltpu.VMEM((B,), jnp.int32),
            pltpu.VMEM((B, 16), jnp.float32),
            pltpu.VMEM((128, 16), jnp.float32),
            pltpu.VMEM_SHARED((N, 16), jnp.float32),
        ],
    )
    def deg_kernel(dst_hbm, out_hbm, dst_v, ones_v, zero_v, acc):
        c = lax.axis_index("c")
        s = lax.axis_index("s")
        _fill(ones_v, B, 16, 1.0)
        _fill(zero_v, 128, 16, 0.0)
        _acc_zero(zero_v, acc, s)
        plsc.subcore_barrier()
        n_t = (nb_sc - s + NS - 1) // NS

        def body(t, carry):
            b = c * nb_sc + t * NS + s
            pltpu.sync_copy(dst_hbm.at[pl.ds(b * B, B)], dst_v)
            pltpu.sync_copy(ones_v, acc.at[dst_v], add=True)
            return carry

        lax.fori_loop(0, n_t, body, 0)
        plsc.subcore_barrier()
        _acc_writeout(acc, out_hbm, c, s)

    return deg_kernel


def _make_agg_kernel(chan_split):
    mesh = plsc.VectorSubcoreMesh(core_axis_name="c", subcore_axis_name="s")
    nb_sc = NB if chan_split else NB // NC

    @functools.partial(
        pl.kernel,
        mesh=mesh,
        out_type=jax.ShapeDtypeStruct((NC * N, 128), jnp.float32),
        scratch_types=[
            pltpu.VMEM((B,), jnp.int32),
            pltpu.VMEM((B,), jnp.int32),
            pltpu.VMEM((B, 128), jnp.float32),
            pltpu.SemaphoreType.DMA,
            pltpu.VMEM_SHARED((N, 128), jnp.float32),
        ],
    )
    def agg_kernel(table_hbm, src_hbm, dst_hbm, out_hbm,
                   sv, dv, rows_v, sem_g, acc):
        c = lax.axis_index("c")
        s = lax.axis_index("s")
        _fill(rows_v, B, 128, 0.0)
        _acc_zero(rows_v, acc, s)
        plsc.subcore_barrier()
        n_t = (nb_sc - s + NS - 1) // NS

        def body(t, carry):
            b = (t * NS + s) if chan_split else (c * nb_sc + t * NS + s)
            pltpu.sync_copy(src_hbm.at[pl.ds(b * B, B)], sv)
            pltpu.sync_copy(dst_hbm.at[pl.ds(b * B, B)], dv)
            if chan_split:
                off = c * N
                for j in range(B // 16):
                    sv[pl.ds(j * 16, 16)] = sv[pl.ds(j * 16, 16)] + off
            pltpu.async_copy(table_hbm.at[sv], rows_v, sem_g).wait()
            pltpu.sync_copy(rows_v, acc.at[dv], add=True)
            return carry

        lax.fori_loop(0, n_t, body, 0)
        plsc.subcore_barrier()
        _acc_writeout(acc, out_hbm, c, s)

    return agg_kernel


_deg_call = _make_deg_kernel()
_agg_l1 = _make_agg_kernel(chan_split=True)
_agg_l2 = _make_agg_kernel(chan_split=False)


def _d_from_degp(degp_ref):
    deg = degp_ref[0, :, 0:1] + degp_ref[1, :, 0:1] + 1.0
    return lax.rsqrt(deg)


def _tc1_body(x_ref, w1_ref, degp_ref, p1_ref):
    h = jnp.dot(x_ref[...], w1_ref[...],
                preferred_element_type=jnp.float32,
                precision=lax.Precision.HIGHEST)
    d = _d_from_degp(degp_ref)
    p1_ref[0] = h[:, :128] * d
    p1_ref[1] = h[:, 128:] * d


def _tc2_body(agg_ref, p1_ref, degp_ref, b1_ref, w2_ref, p2_ref):
    d = _d_from_degp(degp_ref)
    s0 = jnp.maximum(d * (agg_ref[0] + p1_ref[0]) + b1_ref[:, :128], 0.0)
    s1 = jnp.maximum(d * (agg_ref[1] + p1_ref[1]) + b1_ref[:, 128:], 0.0)
    h2 = (jnp.dot(s0, w2_ref[:128, :], preferred_element_type=jnp.float32,
                  precision=lax.Precision.HIGHEST)
          + jnp.dot(s1, w2_ref[128:, :], preferred_element_type=jnp.float32,
                    precision=lax.Precision.HIGHEST))
    p2_ref[...] = d * h2


def _tc3_body(agg_ref, p2_ref, degp_ref, b2_ref, out_ref):
    d = _d_from_degp(degp_ref)
    out_ref[...] = d * (agg_ref[0] + agg_ref[1] + p2_ref[...]) + b2_ref[...]


_RB = 1000
_GRID = N // _RB

_degp_spec = pl.BlockSpec((2, _RB, 16), lambda i: (0, i, 0))


def _tc1(x, w1, degp):
    return pl.pallas_call(
        _tc1_body,
        grid=(_GRID,),
        in_specs=[
            pl.BlockSpec((_RB, 128), lambda i: (i, 0)),
            pl.BlockSpec((128, 256), lambda i: (0, 0)),
            _degp_spec,
        ],
        out_specs=pl.BlockSpec((2, _RB, 128), lambda i: (0, i, 0)),
        out_shape=jax.ShapeDtypeStruct((2, N, 128), jnp.float32),
    )(x, w1, degp)


def _tc2(agg1, p1, degp, b1, w2):
    return pl.pallas_call(
        _tc2_body,
        grid=(_GRID,),
        in_specs=[
            pl.BlockSpec((2, _RB, 128), lambda i: (0, i, 0)),
            pl.BlockSpec((2, _RB, 128), lambda i: (0, i, 0)),
            _degp_spec,
            pl.BlockSpec((1, 256), lambda i: (0, 0)),
            pl.BlockSpec((256, 128), lambda i: (0, 0)),
        ],
        out_specs=pl.BlockSpec((_RB, 128), lambda i: (i, 0)),
        out_shape=jax.ShapeDtypeStruct((N, 128), jnp.float32),
    )(agg1, p1, degp, b1, w2)


def _tc3(agg2, p2, degp, b2):
    return pl.pallas_call(
        _tc3_body,
        grid=(_GRID,),
        in_specs=[
            pl.BlockSpec((2, _RB, 128), lambda i: (0, i, 0)),
            pl.BlockSpec((_RB, 128), lambda i: (i, 0)),
            _degp_spec,
            pl.BlockSpec((1, 128), lambda i: (0, 0)),
        ],
        out_specs=pl.BlockSpec((_RB, 128), lambda i: (i, 0)),
        out_shape=jax.ShapeDtypeStruct((N, 128), jnp.float32),
    )(agg2, p2, degp, b2)


def kernel(x, edge_index, W1, b1, W2, b2):
    ei = edge_index.astype(jnp.int32)
    src, dst = ei[0], ei[1]

    degp = _deg_call(dst).reshape(2, N, 16)
    p1 = _tc1(x, W1, degp)
    agg1 = _agg_l1(p1.reshape(2 * N, 128), src, dst).reshape(2, N, 128)
    p2 = _tc2(agg1, p1, degp, b1.reshape(1, 256), W2)
    agg2 = _agg_l2(p2, src, dst).reshape(2, N, 128)
    return _tc3(agg2, p2, degp, b2.reshape(1, 128))
